# baseline (speedup 1.0000x reference)
import dataclasses
import functools

import jax
import jax.numpy as jnp
from jax import lax
from jax.experimental import pallas as pl
from jax.experimental.pallas import tpu as pltpu
from jax.experimental.pallas import tpu_sc as plsc

_NC, _NS, _L = 2, 16, 16
_NW = _NC * _NS


def _uv_render_sc(vvv, bary_flat, vt_to_v, fuv_flat, pf_flat, *, B, V, D, F, H, W):
    P = H * W
    C = 32
    PW = P // _NW
    NCHUNK = PW // C
    ROWS_PER_H = W // C
    BD = B * D
    NVTP = vt_to_v.shape[0]
    NF3 = fuv_flat.shape[0]
    mesh = plsc.VectorSubcoreMesh(core_axis_name="c", subcore_axis_name="s")
    cp = pltpu.CompilerParams()
    if "needs_layout_passes" in pltpu.CompilerParams.__dataclass_fields__:
        cp = dataclasses.replace(cp, needs_layout_passes=False)
    if "use_tc_tiling_on_sc" in pltpu.CompilerParams.__dataclass_fields__:
        cp = dataclasses.replace(cp, use_tc_tiling_on_sc=False)

    @functools.partial(
        pl.kernel,
        compiler_params=cp,
        out_type=jax.ShapeDtypeStruct((B, H, D, W), jnp.float32),
        mesh=mesh,
        scratch_types=[
            pltpu.VMEM((NVTP,), jnp.int32),
            pltpu.VMEM((NF3,), jnp.int32),
            pltpu.VMEM((PW,), jnp.int32),
            pltpu.VMEM((16 + 3 * PW,), jnp.float32),
            pltpu.VMEM((2, 3 * C), jnp.int32),
            pltpu.VMEM((2 * 3 * C, BD), jnp.float32),
            pltpu.VMEM((2, B, D, C), jnp.float32),
            pltpu.SemaphoreType.DMA,
            pltpu.SemaphoreType.DMA,
            pltpu.SemaphoreType.DMA,
            pltpu.SemaphoreType.DMA,
            pltpu.SemaphoreType.DMA,
        ],
    )
    def k(vvv_hbm, bary_hbm, vt_hbm, fuv_hbm, pf_hbm, out_hbm,
          vt_v, fuv_v, pf_all, bary_all, idx_v, gbuf, obuf,
          sem_in, sem_g0, sem_g1, sem_o0, sem_o1):
        wid = lax.axis_index("s") * _NC + lax.axis_index("c")
        iota = lax.iota(jnp.int32, _L)

        def splat(v):
            return jnp.full((_L,), v, jnp.int32)

        ins = [
            pltpu.async_copy(vt_hbm, vt_v, sem_in),
            pltpu.async_copy(fuv_hbm, fuv_v, sem_in),
            pltpu.async_copy(pf_hbm.at[pl.ds(wid * PW, PW)], pf_all, sem_in),
            pltpu.async_copy(bary_hbm.at[pl.ds(wid * 3 * PW, 3 * PW)],
                             bary_all.at[pl.ds(16, 3 * PW)], sem_in),
        ]
        for h_ in ins:
            h_.wait()

        def gather_copy(par, sem):
            return pltpu.make_async_copy(
                vvv_hbm.at[idx_v.at[par]],
                gbuf.at[pl.ds(par * 3 * C, 3 * C)], sem)

        def fire(c, par):
            sem = sem_g0 if par == 0 else sem_g1
            for g in range(C // _L):
                pfv = plsc.load_gather(pf_all, [splat(c * C + g * _L) + iota])
                pfv = jnp.minimum(jnp.maximum(pfv, 0), F - 1)
                for kk in range(3):
                    uv = plsc.load_gather(fuv_v, [pfv * 3 + kk])
                    vi = plsc.load_gather(vt_v, [uv])
                    plsc.store_scatter(
                        idx_v, [splat(par), splat(kk * C + g * _L) + iota], vi)
            gather_copy(par, sem).start()

        def drain(par):
            sem = sem_g0 if par == 0 else sem_g1
            gather_copy(par, sem).wait()

        def combine(c, par):
            @pl.loop(0, C)
            def _(p):
                wbase = 16 + (c * C + p) * 3
                w = [plsc.load_gather(bary_all, [splat(wbase + kk)])
                     for kk in range(3)]
                for b in range(B):
                    for h in range(D // _L):
                        lane = iota + (b * D + h * _L)
                        r = [plsc.load_gather(
                                gbuf, [splat(par * 3 * C + kk * C + p), lane])
                             for kk in range(3)]
                        acc = w[0] * r[0] + w[1] * r[1] + w[2] * r[2]
                        plsc.store_scatter(
                            obuf,
                            [splat(par), splat(b), iota + h * _L, splat(p)],
                            acc)

        def out_copies(c, par, sem):
            hrow = wid * (PW // W) + c // ROWS_PER_H
            w0 = (c % ROWS_PER_H) * C
            return [
                pltpu.make_async_copy(
                    obuf.at[par, b],
                    out_hbm.at[b, hrow, :, pl.ds(w0, C)], sem)
                for b in range(B)
            ]

        fire(0, 0)

        @pl.loop(0, NCHUNK, step=2)
        def _(c0):
            fire(c0 + 1, 1)
            drain(0)

            @pl.when(c0 >= 2)
            def _():
                for cpd in out_copies(c0 - 2, 0, sem_o0):
                    cpd.wait()

            combine(c0, 0)
            for cpd in out_copies(c0, 0, sem_o0):
                cpd.start()

            @pl.when(c0 + 2 < NCHUNK)
            def _():
                fire(c0 + 2, 0)

            drain(1)

            @pl.when(c0 >= 2)
            def _():
                for cpd in out_copies(c0 - 1, 1, sem_o1):
                    cpd.wait()

            combine(c0 + 1, 1)
            for cpd in out_copies(c0 + 1, 1, sem_o1):
                cpd.start()

        for cpd in out_copies(NCHUNK - 2, 0, sem_o0):
            cpd.wait()
        for cpd in out_copies(NCHUNK - 1, 1, sem_o1):
            cpd.wait()

    return k(vvv, bary_flat, vt_to_v, fuv_flat, pf_flat)


def kernel(verts_attr, bary_coords, vt_to_v_index, faces_uv, pix_to_face):
    B, V, D = verts_attr.shape
    F = faces_uv.shape[0]
    H, W = pix_to_face.shape
    P = H * W
    vt = vt_to_v_index.astype(jnp.int32)
    vt_pad = (-vt.shape[0]) % 16
    if vt_pad:
        vt = jnp.pad(vt, (0, vt_pad))
    out = _uv_render_sc(
        verts_attr.transpose(1, 0, 2).reshape(V, B * D),
        bary_coords.astype(jnp.float32).reshape(P * 3),
        vt,
        faces_uv.astype(jnp.int32).reshape(F * 3),
        pix_to_face.astype(jnp.int32).reshape(P),
        B=B, V=V, D=D, F=F, H=H, W=W,
    )
    return out.swapaxes(2, 3)


# speedup vs baseline: 1.1240x; 1.1240x over previous
import dataclasses
import functools

import jax
import jax.numpy as jnp
from jax import lax
from jax.experimental import pallas as pl
from jax.experimental.pallas import tpu as pltpu
from jax.experimental.pallas import tpu_sc as plsc

_NC, _NS, _L = 2, 16, 16
_NW = _NC * _NS


def _uv_render_sc(vvv, bary_flat, vt_to_v, fuv_flat, pf_flat, *, B, V, D, F, H, W):
    P = H * W
    C = 32
    PW = P // _NW
    NCHUNK = PW // C
    ROWS_PER_H = W // C
    BD = B * D
    NVTP = vt_to_v.shape[0]
    NF3 = fuv_flat.shape[0]
    mesh = plsc.VectorSubcoreMesh(core_axis_name="c", subcore_axis_name="s")
    cp = pltpu.CompilerParams()
    if "needs_layout_passes" in pltpu.CompilerParams.__dataclass_fields__:
        cp = dataclasses.replace(cp, needs_layout_passes=False)
    if "use_tc_tiling_on_sc" in pltpu.CompilerParams.__dataclass_fields__:
        cp = dataclasses.replace(cp, use_tc_tiling_on_sc=False)

    @functools.partial(
        pl.kernel,
        compiler_params=cp,
        out_type=jax.ShapeDtypeStruct((B, H, W, D), jnp.float32),
        mesh=mesh,
        scratch_types=[
            pltpu.VMEM((NVTP,), jnp.int32),
            pltpu.VMEM((NF3,), jnp.int32),
            pltpu.VMEM((PW,), jnp.int32),
            pltpu.VMEM((16 + 3 * PW,), jnp.float32),
            pltpu.VMEM((2, 3 * C), jnp.int32),
            pltpu.VMEM((2 * 3 * C, BD), jnp.float32),
            pltpu.VMEM((2, B, C, D), jnp.float32),
            pltpu.SemaphoreType.DMA,
            pltpu.SemaphoreType.DMA,
            pltpu.SemaphoreType.DMA,
            pltpu.SemaphoreType.DMA,
            pltpu.SemaphoreType.DMA,
        ],
    )
    def k(vvv_hbm, bary_hbm, vt_hbm, fuv_hbm, pf_hbm, out_hbm,
          vt_v, fuv_v, pf_all, bary_all, idx_v, gbuf, obuf,
          sem_in, sem_g0, sem_g1, sem_o0, sem_o1):
        wid = lax.axis_index("s") * _NC + lax.axis_index("c")
        iota = lax.iota(jnp.int32, _L)

        def splat(v):
            return jnp.full((_L,), v, jnp.int32)

        ins = [
            pltpu.async_copy(vt_hbm, vt_v, sem_in),
            pltpu.async_copy(fuv_hbm, fuv_v, sem_in),
            pltpu.async_copy(pf_hbm.at[pl.ds(wid * PW, PW)], pf_all, sem_in),
            pltpu.async_copy(bary_hbm.at[pl.ds(wid * 3 * PW, 3 * PW)],
                             bary_all.at[pl.ds(16, 3 * PW)], sem_in),
        ]
        for h_ in ins:
            h_.wait()

        def gather_copy(par, sem):
            return pltpu.make_async_copy(
                vvv_hbm.at[idx_v.at[par]],
                gbuf.at[pl.ds(par * 3 * C, 3 * C)], sem)

        def fire(c, par):
            sem = sem_g0 if par == 0 else sem_g1
            for g in range(C // _L):
                pfv = plsc.load_gather(pf_all, [splat(c * C + g * _L) + iota])
                pfv = jnp.minimum(jnp.maximum(pfv, 0), F - 1)
                for kk in range(3):
                    uv = plsc.load_gather(fuv_v, [pfv * 3 + kk])
                    vi = plsc.load_gather(vt_v, [uv])
                    plsc.store_scatter(
                        idx_v, [splat(par), splat(kk * C + g * _L) + iota], vi)
            gather_copy(par, sem).start()

        def drain(par):
            sem = sem_g0 if par == 0 else sem_g1
            gather_copy(par, sem).wait()

        def combine(c, par):
            @pl.loop(0, C)
            def _(p):
                wbase = 16 + (c * C + p) * 3
                w = [plsc.load_gather(bary_all, [splat(wbase + kk)])
                     for kk in range(3)]
                for b in range(B):
                    for h in range(D // _L):
                        lane = iota + (b * D + h * _L)
                        r = [plsc.load_gather(
                                gbuf, [splat(par * 3 * C + kk * C + p), lane])
                             for kk in range(3)]
                        acc = w[0] * r[0] + w[1] * r[1] + w[2] * r[2]
                        plsc.store_scatter(
                            obuf,
                            [splat(par), splat(b), splat(p), iota + h * _L],
                            acc)

        def out_copies(c, par, sem):
            hrow = wid * (PW // W) + c // ROWS_PER_H
            w0 = (c % ROWS_PER_H) * C
            return [
                pltpu.make_async_copy(
                    obuf.at[par, b],
                    out_hbm.at[b, hrow, pl.ds(w0, C), :], sem)
                for b in range(B)
            ]

        fire(0, 0)

        @pl.loop(0, NCHUNK, step=2)
        def _(c0):
            fire(c0 + 1, 1)
            drain(0)

            @pl.when(c0 >= 2)
            def _():
                for cpd in out_copies(c0 - 2, 0, sem_o0):
                    cpd.wait()

            combine(c0, 0)
            for cpd in out_copies(c0, 0, sem_o0):
                cpd.start()

            @pl.when(c0 + 2 < NCHUNK)
            def _():
                fire(c0 + 2, 0)

            drain(1)

            @pl.when(c0 >= 2)
            def _():
                for cpd in out_copies(c0 - 1, 1, sem_o1):
                    cpd.wait()

            combine(c0 + 1, 1)
            for cpd in out_copies(c0 + 1, 1, sem_o1):
                cpd.start()

        for cpd in out_copies(NCHUNK - 2, 0, sem_o0):
            cpd.wait()
        for cpd in out_copies(NCHUNK - 1, 1, sem_o1):
            cpd.wait()

    return k(vvv, bary_flat, vt_to_v, fuv_flat, pf_flat)


def kernel(verts_attr, bary_coords, vt_to_v_index, faces_uv, pix_to_face):
    B, V, D = verts_attr.shape
    F = faces_uv.shape[0]
    H, W = pix_to_face.shape
    P = H * W
    vt = vt_to_v_index.astype(jnp.int32)
    vt_pad = (-vt.shape[0]) % 16
    if vt_pad:
        vt = jnp.pad(vt, (0, vt_pad))
    out = _uv_render_sc(
        verts_attr.transpose(1, 0, 2).reshape(V, B * D),
        bary_coords.astype(jnp.float32).reshape(P * 3),
        vt,
        faces_uv.astype(jnp.int32).reshape(F * 3),
        pix_to_face.astype(jnp.int32).reshape(P),
        B=B, V=V, D=D, F=F, H=H, W=W,
    )
    return out


# speedup vs baseline: 1.4307x; 1.2729x over previous
import dataclasses
import functools

import jax
import jax.numpy as jnp
from jax import lax
from jax.experimental import pallas as pl
from jax.experimental.pallas import tpu as pltpu
from jax.experimental.pallas import tpu_sc as plsc

_NC, _NS, _L = 2, 16, 16
_NW = _NC * _NS


def _uv_render_sc(vvv, bary_p, vt_to_v, fuv_c, pf_flat, *, B, V, D, F, H, W):
    P = H * W
    C = 32
    OPITCH = 40
    PW = P // _NW
    NCHUNK = PW // C
    ROWS_PER_H = W // C
    BD = B * D
    NVTP = vt_to_v.shape[0]
    mesh = plsc.VectorSubcoreMesh(core_axis_name="c", subcore_axis_name="s")
    cp = pltpu.CompilerParams()
    if "needs_layout_passes" in pltpu.CompilerParams.__dataclass_fields__:
        cp = dataclasses.replace(cp, needs_layout_passes=False)
    if "use_tc_tiling_on_sc" in pltpu.CompilerParams.__dataclass_fields__:
        cp = dataclasses.replace(cp, use_tc_tiling_on_sc=False)

    @functools.partial(
        pl.kernel,
        compiler_params=cp,
        out_type=jax.ShapeDtypeStruct((B, H, D, W), jnp.float32),
        mesh=mesh,
        scratch_types=[
            pltpu.VMEM((NVTP,), jnp.int32),
            pltpu.VMEM((3, F), jnp.int32),
            pltpu.VMEM((PW,), jnp.int32),
            pltpu.VMEM((3, 16 + PW), jnp.float32),
            pltpu.VMEM((2, 3 * C), jnp.int32),
            pltpu.VMEM((2 * 3 * C, BD), jnp.float32),
            pltpu.VMEM((2, B, D, OPITCH), jnp.float32),
            pltpu.SemaphoreType.DMA,
            pltpu.SemaphoreType.DMA,
            pltpu.SemaphoreType.DMA,
            pltpu.SemaphoreType.DMA,
            pltpu.SemaphoreType.DMA,
        ],
    )
    def k(vvv_hbm, bary0_hbm, bary1_hbm, bary2_hbm, vt_hbm,
          fuv0_hbm, fuv1_hbm, fuv2_hbm, pf_hbm, out_hbm,
          vt_v, fuv_v, pf_all, bary_v, idx_v, gbuf, obuf,
          sem_in, sem_g0, sem_g1, sem_o0, sem_o1):
        wid = lax.axis_index("s") * _NC + lax.axis_index("c")
        iota = lax.iota(jnp.int32, _L)
        bary_hbms = [bary0_hbm, bary1_hbm, bary2_hbm]
        fuv_hbms = [fuv0_hbm, fuv1_hbm, fuv2_hbm]

        def splat(v):
            return jnp.full((_L,), v, jnp.int32)

        ins = [
            pltpu.async_copy(vt_hbm, vt_v, sem_in),
            pltpu.async_copy(pf_hbm.at[pl.ds(wid * PW, PW)], pf_all, sem_in),
        ]
        ins += [pltpu.async_copy(fuv_hbms[kk], fuv_v.at[kk], sem_in)
                for kk in range(3)]
        ins += [pltpu.async_copy(bary_hbms[kk].at[pl.ds(wid * PW, PW)],
                                 bary_v.at[kk, pl.ds(16, PW)], sem_in)
                for kk in range(3)]
        for h_ in ins:
            h_.wait()

        def gather_copies(par, sem):
            return [
                pltpu.make_async_copy(
                    vvv_hbm.at[idx_v.at[par, pl.ds(48 * i, 48)]],
                    gbuf.at[pl.ds(par * 3 * C + 48 * i, 48)], sem)
                for i in range(2)
            ]

        def fire(c, par):
            sem = sem_g0 if par == 0 else sem_g1
            for g in range(C // _L):
                pfv = plsc.load_gather(pf_all, [splat(c * C + g * _L) + iota])
                pfv = jnp.minimum(jnp.maximum(pfv, 0), F - 1)
                for kk in range(3):
                    uv = plsc.load_gather(fuv_v.at[kk], [pfv])
                    vi = plsc.load_gather(vt_v, [uv])
                    plsc.store_scatter(
                        idx_v, [splat(par), splat(kk * C + g * _L) + iota], vi)
            for cpd in gather_copies(par, sem):
                cpd.start()

        def drain(par):
            sem = sem_g0 if par == 0 else sem_g1
            for cpd in gather_copies(par, sem):
                cpd.wait()

        def combine(c, par):
            @pl.loop(0, C)
            def _(p):
                wbase = 16 + c * C + p
                w = [plsc.load_gather(bary_v.at[kk], [splat(wbase)])
                     for kk in range(3)]
                for b in range(B):
                    for h in range(D // _L):
                        lane = iota + (b * D + h * _L)
                        r = [plsc.load_gather(
                                gbuf, [splat(par * 3 * C + kk * C + p), lane])
                             for kk in range(3)]
                        acc = w[0] * r[0] + w[1] * r[1] + w[2] * r[2]
                        plsc.store_scatter(
                            obuf,
                            [splat(par), splat(b), iota + h * _L, splat(p)],
                            acc)

        def out_copies(c, par, sem):
            hrow = wid * (PW // W) + c // ROWS_PER_H
            w0 = (c % ROWS_PER_H) * C
            return [
                pltpu.make_async_copy(
                    obuf.at[par, b, :, pl.ds(0, C)],
                    out_hbm.at[b, hrow, :, pl.ds(w0, C)], sem)
                for b in range(B)
            ]

        fire(0, 0)

        @pl.loop(0, NCHUNK, step=2)
        def _(c0):
            fire(c0 + 1, 1)
            drain(0)

            @pl.when(c0 >= 2)
            def _():
                for cpd in out_copies(c0 - 2, 0, sem_o0):
                    cpd.wait()

            combine(c0, 0)
            for cpd in out_copies(c0, 0, sem_o0):
                cpd.start()

            @pl.when(c0 + 2 < NCHUNK)
            def _():
                fire(c0 + 2, 0)

            drain(1)

            @pl.when(c0 >= 2)
            def _():
                for cpd in out_copies(c0 - 1, 1, sem_o1):
                    cpd.wait()

            combine(c0 + 1, 1)
            for cpd in out_copies(c0 + 1, 1, sem_o1):
                cpd.start()

        for cpd in out_copies(NCHUNK - 2, 0, sem_o0):
            cpd.wait()
        for cpd in out_copies(NCHUNK - 1, 1, sem_o1):
            cpd.wait()

    return k(vvv, bary_p[0], bary_p[1], bary_p[2], vt_to_v,
             fuv_c[0], fuv_c[1], fuv_c[2], pf_flat)


def kernel(verts_attr, bary_coords, vt_to_v_index, faces_uv, pix_to_face):
    B, V, D = verts_attr.shape
    F = faces_uv.shape[0]
    H, W = pix_to_face.shape
    P = H * W
    vt = vt_to_v_index.astype(jnp.int32)
    vt_pad = (-vt.shape[0]) % 16
    if vt_pad:
        vt = jnp.pad(vt, (0, vt_pad))
    bary = bary_coords.astype(jnp.float32)
    fuv = faces_uv.astype(jnp.int32)
    out = _uv_render_sc(
        verts_attr.transpose(1, 0, 2).reshape(V, B * D),
        [bary[:, :, kk].reshape(P) for kk in range(3)],
        vt,
        [fuv[:, kk] for kk in range(3)],
        pix_to_face.astype(jnp.int32).reshape(P),
        B=B, V=V, D=D, F=F, H=H, W=W,
    )
    return out.swapaxes(2, 3)


# speedup vs baseline: 1.4321x; 1.0009x over previous
import dataclasses
import functools

import jax
import jax.numpy as jnp
from jax import lax
from jax.experimental import pallas as pl
from jax.experimental.pallas import tpu as pltpu
from jax.experimental.pallas import tpu_sc as plsc

_NC, _NS, _L = 2, 16, 16
_NW = _NC * _NS


def _uv_render_sc(vvv, bary_p, vt_to_v, fuv_c, pf_flat, *, B, V, D, F, H, W):
    P = H * W
    C = 32
    OPITCH = 40
    PW = P // _NW
    NCHUNK = PW // C
    ROWS_PER_H = W // C
    BD = B * D
    NVTP = vt_to_v.shape[0]
    mesh = plsc.VectorSubcoreMesh(core_axis_name="c", subcore_axis_name="s")
    cp = pltpu.CompilerParams()
    if "needs_layout_passes" in pltpu.CompilerParams.__dataclass_fields__:
        cp = dataclasses.replace(cp, needs_layout_passes=False)
    if "use_tc_tiling_on_sc" in pltpu.CompilerParams.__dataclass_fields__:
        cp = dataclasses.replace(cp, use_tc_tiling_on_sc=False)

    @functools.partial(
        pl.kernel,
        compiler_params=cp,
        out_type=jax.ShapeDtypeStruct((B, H, D, W), jnp.float32),
        mesh=mesh,
        scratch_types=[
            pltpu.VMEM((NVTP,), jnp.int32),
            pltpu.VMEM((3, F), jnp.int32),
            pltpu.VMEM((PW,), jnp.int32),
            pltpu.VMEM((3, 16 + PW), jnp.float32),
            pltpu.VMEM((2, 3 * C), jnp.int32),
            pltpu.VMEM((2 * 3 * C, BD), jnp.float32),
            pltpu.VMEM((2, B, D, OPITCH), jnp.float32),
            pltpu.SemaphoreType.DMA,
            pltpu.SemaphoreType.DMA,
            pltpu.SemaphoreType.DMA,
            pltpu.SemaphoreType.DMA,
            pltpu.SemaphoreType.DMA,
        ],
    )
    def k(vvv_hbm, bary0_hbm, bary1_hbm, bary2_hbm, vt_hbm,
          fuv0_hbm, fuv1_hbm, fuv2_hbm, pf_hbm, out_hbm,
          vt_v, fuv_v, pf_all, bary_v, idx_v, gbuf, obuf,
          sem_in, sem_g0, sem_g1, sem_o0, sem_o1):
        wid = lax.axis_index("s") * _NC + lax.axis_index("c")
        iota = lax.iota(jnp.int32, _L)
        bary_hbms = [bary0_hbm, bary1_hbm, bary2_hbm]
        fuv_hbms = [fuv0_hbm, fuv1_hbm, fuv2_hbm]

        def splat(v):
            return jnp.full((_L,), v, jnp.int32)

        lane_c = [iota + (b * D + h * _L) for b in range(B) for h in range(D // _L)]
        dvec_c = [iota + h * _L for h in range(D // _L)]
        b_c = [splat(b) for b in range(B)]
        par_c = [splat(0), splat(1)]

        ins = [
            pltpu.async_copy(vt_hbm, vt_v, sem_in),
            pltpu.async_copy(pf_hbm.at[pl.ds(wid * PW, PW)], pf_all, sem_in),
        ]
        ins += [pltpu.async_copy(fuv_hbms[kk], fuv_v.at[kk], sem_in)
                for kk in range(3)]
        ins += [pltpu.async_copy(bary_hbms[kk].at[pl.ds(wid * PW, PW)],
                                 bary_v.at[kk, pl.ds(16, PW)], sem_in)
                for kk in range(3)]
        for h_ in ins:
            h_.wait()

        def gather_copies(par, sem):
            return [
                pltpu.make_async_copy(
                    vvv_hbm.at[idx_v.at[par, pl.ds(48 * i, 48)]],
                    gbuf.at[pl.ds(par * 3 * C + 48 * i, 48)], sem)
                for i in range(2)
            ]

        def fire(c, par):
            sem = sem_g0 if par == 0 else sem_g1
            for g in range(C // _L):
                pfv = plsc.load_gather(pf_all, [splat(c * C + g * _L) + iota])
                pfv = jnp.minimum(jnp.maximum(pfv, 0), F - 1)
                for kk in range(3):
                    uv = plsc.load_gather(fuv_v.at[kk], [pfv])
                    vi = plsc.load_gather(vt_v, [uv])
                    plsc.store_scatter(
                        idx_v, [splat(par), splat(kk * C + g * _L) + iota], vi)
            for cpd in gather_copies(par, sem):
                cpd.start()

        def drain(par):
            sem = sem_g0 if par == 0 else sem_g1
            for cpd in gather_copies(par, sem):
                cpd.wait()

        def combine(c, par):
            @pl.loop(0, C)
            def _(p):
                wbase = splat(16 + c * C + p)
                w = [plsc.load_gather(bary_v.at[kk], [wbase])
                     for kk in range(3)]
                rows = [splat(par * 3 * C + kk * C + p) for kk in range(3)]
                pvec = splat(p)
                for b in range(B):
                    for h in range(D // _L):
                        r = [plsc.load_gather(
                                gbuf, [rows[kk], lane_c[b * (D // _L) + h]])
                             for kk in range(3)]
                        acc = w[0] * r[0] + w[1] * r[1] + w[2] * r[2]
                        plsc.store_scatter(
                            obuf,
                            [par_c[par], b_c[b], dvec_c[h], pvec],
                            acc)

        def out_copies(c, par, sem):
            hrow = wid * (PW // W) + c // ROWS_PER_H
            w0 = (c % ROWS_PER_H) * C
            return [
                pltpu.make_async_copy(
                    obuf.at[par, b, :, pl.ds(0, C)],
                    out_hbm.at[b, hrow, :, pl.ds(w0, C)], sem)
                for b in range(B)
            ]

        fire(0, 0)

        @pl.loop(0, NCHUNK, step=2)
        def _(c0):
            fire(c0 + 1, 1)
            drain(0)

            @pl.when(c0 >= 2)
            def _():
                for cpd in out_copies(c0 - 2, 0, sem_o0):
                    cpd.wait()

            combine(c0, 0)
            for cpd in out_copies(c0, 0, sem_o0):
                cpd.start()

            @pl.when(c0 + 2 < NCHUNK)
            def _():
                fire(c0 + 2, 0)

            drain(1)

            @pl.when(c0 >= 2)
            def _():
                for cpd in out_copies(c0 - 1, 1, sem_o1):
                    cpd.wait()

            combine(c0 + 1, 1)
            for cpd in out_copies(c0 + 1, 1, sem_o1):
                cpd.start()

        for cpd in out_copies(NCHUNK - 2, 0, sem_o0):
            cpd.wait()
        for cpd in out_copies(NCHUNK - 1, 1, sem_o1):
            cpd.wait()

    return k(vvv, bary_p[0], bary_p[1], bary_p[2], vt_to_v,
             fuv_c[0], fuv_c[1], fuv_c[2], pf_flat)


def kernel(verts_attr, bary_coords, vt_to_v_index, faces_uv, pix_to_face):
    B, V, D = verts_attr.shape
    F = faces_uv.shape[0]
    H, W = pix_to_face.shape
    P = H * W
    vt = vt_to_v_index.astype(jnp.int32)
    vt_pad = (-vt.shape[0]) % 16
    if vt_pad:
        vt = jnp.pad(vt, (0, vt_pad))
    bary = bary_coords.astype(jnp.float32)
    fuv = faces_uv.astype(jnp.int32)
    out = _uv_render_sc(
        verts_attr.transpose(1, 0, 2).reshape(V, B * D),
        [bary[:, :, kk].reshape(P) for kk in range(3)],
        vt,
        [fuv[:, kk] for kk in range(3)],
        pix_to_face.astype(jnp.int32).reshape(P),
        B=B, V=V, D=D, F=F, H=H, W=W,
    )
    return out.swapaxes(2, 3)


# speedup vs baseline: 2.1347x; 1.4906x over previous
import dataclasses
import functools

import jax
import jax.numpy as jnp
from jax import lax
from jax.experimental import pallas as pl
from jax.experimental.pallas import tpu as pltpu
from jax.experimental.pallas import tpu_sc as plsc

_NC, _NS, _L = 2, 16, 16
_NW = _NC * _NS


def _uv_render_sc(vvv, bary_p, vt_to_v, fuv_c, pf_flat, *, B, V, D, F, H, W):
    P = H * W
    C = 32
    OPITCH = 40
    PW = P // _NW
    NCHUNK = PW // C
    ROWS_PER_H = W // C
    BD = B * D
    NVTP = vt_to_v.shape[0]
    mesh = plsc.VectorSubcoreMesh(core_axis_name="c", subcore_axis_name="s")
    cp = pltpu.CompilerParams()
    if "needs_layout_passes" in pltpu.CompilerParams.__dataclass_fields__:
        cp = dataclasses.replace(cp, needs_layout_passes=False)
    if "use_tc_tiling_on_sc" in pltpu.CompilerParams.__dataclass_fields__:
        cp = dataclasses.replace(cp, use_tc_tiling_on_sc=False)

    @functools.partial(
        pl.kernel,
        compiler_params=cp,
        out_type=jax.ShapeDtypeStruct((B, H, D, W), jnp.float32),
        mesh=mesh,
        scratch_types=[
            pltpu.VMEM((NVTP,), jnp.int32),
            pltpu.VMEM((3, F), jnp.int32),
            pltpu.VMEM((PW,), jnp.int32),
            pltpu.VMEM((3, 16 + PW), jnp.float32),
            pltpu.VMEM((2, 3 * C), jnp.int32),
            pltpu.VMEM((2 * 3 * C, BD), jnp.float32),
            pltpu.VMEM((2, B, D, OPITCH), jnp.float32),
            pltpu.SemaphoreType.DMA,
            pltpu.SemaphoreType.DMA,
            pltpu.SemaphoreType.DMA,
            pltpu.SemaphoreType.DMA,
            pltpu.SemaphoreType.DMA,
        ],
    )
    def k(vvv_hbm, bary0_hbm, bary1_hbm, bary2_hbm, vt_hbm,
          fuv0_hbm, fuv1_hbm, fuv2_hbm, pf_hbm, out_hbm,
          vt_v, fuv_v, pf_all, bary_v, idx_v, gbuf, obuf,
          sem_in, sem_g0, sem_g1, sem_o0, sem_o1):
        wid = lax.axis_index("s") * _NC + lax.axis_index("c")
        iota = lax.iota(jnp.int32, _L)
        bary_hbms = [bary0_hbm, bary1_hbm, bary2_hbm]
        fuv_hbms = [fuv0_hbm, fuv1_hbm, fuv2_hbm]

        def splat(v):
            return jnp.full((_L,), v, jnp.int32)

        lane_c = [iota + (b * D + h * _L) for b in range(B) for h in range(D // _L)]
        dvec_c = [iota + h * _L for h in range(D // _L)]
        b_c = [splat(b) for b in range(B)]
        par_c = [splat(0), splat(1)]

        ins = [
            pltpu.async_copy(vt_hbm, vt_v, sem_in),
            pltpu.async_copy(pf_hbm.at[pl.ds(wid * PW, PW)], pf_all, sem_in),
        ]
        ins += [pltpu.async_copy(fuv_hbms[kk], fuv_v.at[kk], sem_in)
                for kk in range(3)]
        ins += [pltpu.async_copy(bary_hbms[kk].at[pl.ds(wid * PW, PW)],
                                 bary_v.at[kk, pl.ds(16, PW)], sem_in)
                for kk in range(3)]
        for h_ in ins:
            h_.wait()

        def gather_copies(par, sem):
            return [
                pltpu.make_async_copy(
                    vvv_hbm.at[idx_v.at[par, pl.ds(48 * i, 48)]],
                    gbuf.at[pl.ds(par * 3 * C + 48 * i, 48)], sem)
                for i in range(2)
            ]

        def fire(c, par):
            sem = sem_g0 if par == 0 else sem_g1
            for g in range(C // _L):
                pfv = plsc.load_gather(pf_all, [splat(c * C + g * _L) + iota])
                pfv = jnp.minimum(jnp.maximum(pfv, 0), F - 1)
                for kk in range(3):
                    uv = plsc.load_gather(fuv_v.at[kk], [pfv])
                    vi = plsc.load_gather(vt_v, [uv])
                    plsc.store_scatter(
                        idx_v, [splat(par), splat(kk * C + g * _L) + iota], vi)
            for cpd in gather_copies(par, sem):
                cpd.start()

        def drain(par):
            sem = sem_g0 if par == 0 else sem_g1
            for cpd in gather_copies(par, sem):
                cpd.wait()

        def combine(c, par):
            @plsc.parallel_loop(0, C, unroll=2)
            def _(p):
                wbase = splat(16 + c * C + p)
                w = [plsc.load_gather(bary_v.at[kk], [wbase])
                     for kk in range(3)]
                rows = [splat(par * 3 * C + kk * C + p) for kk in range(3)]
                pvec = splat(p)
                for b in range(B):
                    for h in range(D // _L):
                        r = [plsc.load_gather(
                                gbuf, [rows[kk], lane_c[b * (D // _L) + h]])
                             for kk in range(3)]
                        acc = w[0] * r[0] + w[1] * r[1] + w[2] * r[2]
                        plsc.store_scatter(
                            obuf,
                            [par_c[par], b_c[b], dvec_c[h], pvec],
                            acc)

        def out_copies(c, par, sem):
            hrow = wid * (PW // W) + c // ROWS_PER_H
            w0 = (c % ROWS_PER_H) * C
            return [
                pltpu.make_async_copy(
                    obuf.at[par, b, :, pl.ds(0, C)],
                    out_hbm.at[b, hrow, :, pl.ds(w0, C)], sem)
                for b in range(B)
            ]

        fire(0, 0)

        @pl.loop(0, NCHUNK, step=2)
        def _(c0):
            fire(c0 + 1, 1)
            drain(0)

            @pl.when(c0 >= 2)
            def _():
                for cpd in out_copies(c0 - 2, 0, sem_o0):
                    cpd.wait()

            combine(c0, 0)
            for cpd in out_copies(c0, 0, sem_o0):
                cpd.start()

            @pl.when(c0 + 2 < NCHUNK)
            def _():
                fire(c0 + 2, 0)

            drain(1)

            @pl.when(c0 >= 2)
            def _():
                for cpd in out_copies(c0 - 1, 1, sem_o1):
                    cpd.wait()

            combine(c0 + 1, 1)
            for cpd in out_copies(c0 + 1, 1, sem_o1):
                cpd.start()

        for cpd in out_copies(NCHUNK - 2, 0, sem_o0):
            cpd.wait()
        for cpd in out_copies(NCHUNK - 1, 1, sem_o1):
            cpd.wait()

    return k(vvv, bary_p[0], bary_p[1], bary_p[2], vt_to_v,
             fuv_c[0], fuv_c[1], fuv_c[2], pf_flat)


def kernel(verts_attr, bary_coords, vt_to_v_index, faces_uv, pix_to_face):
    B, V, D = verts_attr.shape
    F = faces_uv.shape[0]
    H, W = pix_to_face.shape
    P = H * W
    vt = vt_to_v_index.astype(jnp.int32)
    vt_pad = (-vt.shape[0]) % 16
    if vt_pad:
        vt = jnp.pad(vt, (0, vt_pad))
    bary = bary_coords.astype(jnp.float32)
    fuv = faces_uv.astype(jnp.int32)
    out = _uv_render_sc(
        verts_attr.transpose(1, 0, 2).reshape(V, B * D),
        [bary[:, :, kk].reshape(P) for kk in range(3)],
        vt,
        [fuv[:, kk] for kk in range(3)],
        pix_to_face.astype(jnp.int32).reshape(P),
        B=B, V=V, D=D, F=F, H=H, W=W,
    )
    return out.swapaxes(2, 3)
